# Initial kernel scaffold; baseline (speedup 1.0000x reference)
import jax
import jax.numpy as jnp
from jax.experimental import pallas as pl


def kernel(x, W_frame_0, W_frame_1):
    raise NotImplementedError("write your pallas kernel here")


# speedup vs baseline: 4.0658x; 4.0658x over previous
import functools

import jax
import jax.numpy as jnp
from jax import lax
from jax.experimental import pallas as pl
from jax.experimental.pallas import tpu as pltpu
from jax.experimental.pallas import tpu_sc as plsc

NUM_CORES = 2
NUM_SUBCORES = 16
NW = NUM_CORES * NUM_SUBCORES

D = 64
B = 4096 * 50
BPW = B // NW
CHUNK = 128
NCHUNK = BPW // CHUNK


_mesh = plsc.VectorSubcoreMesh(core_axis_name="c", subcore_axis_name="s")


@functools.partial(
    pl.kernel,
    out_type=jax.ShapeDtypeStruct((B, D), jnp.float32),
    mesh=_mesh,
    scratch_types=[
        pltpu.VMEM((BPW,), jnp.int32),
        pltpu.VMEM((2, CHUNK, D), jnp.float32),
        pltpu.SemaphoreType.DMA,
        pltpu.SemaphoreType.DMA,
    ],
    compiler_params=pltpu.CompilerParams(use_tc_tiling_on_sc=False),
)
def _gather_kernel(table, xf, out, idx_v, rows_v, sem_a, sem_b):
    sems = [sem_a, sem_b]
    wid = lax.axis_index("s") * NUM_CORES + lax.axis_index("c")
    base = wid * BPW

    pltpu.sync_copy(xf.at[pl.ds(base, BPW)], idx_v)

    def idx_slice(j):
        return idx_v.at[pl.ds(pl.multiple_of(j * CHUNK, CHUNK), CHUNK)]

    def start(j, slot):
        pltpu.async_copy(table.at[idx_slice(j)], rows_v.at[slot], sems[slot])

    def wait(j, slot):
        pltpu.make_async_copy(
            table.at[idx_slice(j)], rows_v.at[slot], sems[slot]
        ).wait()

    start(0, 0)

    def outer(g, carry):
        for b in range(2):
            j = g * 2 + b

            @pl.when(j + 1 < NCHUNK)
            def _():
                start(j + 1, (b + 1) % 2)

            wait(j, b)
            pltpu.sync_copy(
                rows_v.at[b],
                out.at[pl.ds(pl.multiple_of(base + j * CHUNK, CHUNK), CHUNK)],
            )
        return carry

    lax.fori_loop(0, NCHUNK // 2, outer, 0)


def kernel(x, W_frame_0, W_frame_1):
    table = jnp.concatenate([W_frame_0, W_frame_1], axis=0)
    xf = x.reshape(-1)
    out = _gather_kernel(table, xf)
    return out.reshape(x.shape[0], x.shape[1], D)


# speedup vs baseline: 4.1475x; 1.0201x over previous
import functools

import jax
import jax.numpy as jnp
from jax import lax
from jax.experimental import pallas as pl
from jax.experimental.pallas import tpu as pltpu
from jax.experimental.pallas import tpu_sc as plsc

NUM_CORES = 2
NUM_SUBCORES = 16
NW = NUM_CORES * NUM_SUBCORES

D = 64
B = 4096 * 50
BPW = B // NW
CHUNK = 128
NCHUNK = BPW // CHUNK


_mesh = plsc.VectorSubcoreMesh(core_axis_name="c", subcore_axis_name="s")

RING = 8
AHEAD = 5


@functools.partial(
    pl.kernel,
    out_type=jax.ShapeDtypeStruct((B, D), jnp.float32),
    mesh=_mesh,
    scratch_types=[
        pltpu.VMEM((BPW,), jnp.int32),
        pltpu.VMEM((RING, CHUNK, D), jnp.float32),
        [pltpu.SemaphoreType.DMA] * RING,
        [pltpu.SemaphoreType.DMA] * RING,
    ],
    compiler_params=pltpu.CompilerParams(use_tc_tiling_on_sc=False),
)
def _gather_kernel(table, xf, out, idx_v, rows_v, gsems, wsems):
    wid = lax.axis_index("s") * NUM_CORES + lax.axis_index("c")
    base = wid * BPW

    pltpu.sync_copy(xf.at[pl.ds(base, BPW)], idx_v)

    def idx_slice(j):
        return idx_v.at[pl.ds(pl.multiple_of(j * CHUNK, CHUNK), CHUNK)]

    def out_slice(j):
        return out.at[pl.ds(pl.multiple_of(base + j * CHUNK, CHUNK), CHUNK)]

    def gather(j, slot):
        return pltpu.make_async_copy(
            table.at[idx_slice(j)], rows_v.at[slot], gsems[slot]
        )

    def writeback(j, slot):
        return pltpu.make_async_copy(rows_v.at[slot], out_slice(j), wsems[slot])

    wb_waited = 0
    for j in range(min(AHEAD, NCHUNK)):
        gather(j, j % RING).start()
    for j in range(NCHUNK):
        slot = j % RING
        gather(j, slot).wait()
        writeback(j, slot).start()
        nxt = j + AHEAD
        if nxt < NCHUNK:
            prev = nxt - RING
            if prev >= 0:
                writeback(prev, prev % RING).wait()
                wb_waited = prev + 1
            gather(nxt, nxt % RING).start()
    for j in range(wb_waited, NCHUNK):
        writeback(j, j % RING).wait()


def kernel(x, W_frame_0, W_frame_1):
    table = jnp.concatenate([W_frame_0, W_frame_1], axis=0)
    xf = x.reshape(-1)
    out = _gather_kernel(table, xf)
    return out.reshape(x.shape[0], x.shape[1], D)
